# Initial kernel scaffold; baseline (speedup 1.0000x reference)
import jax
import jax.numpy as jnp
from jax.experimental import pallas as pl


def kernel(input_ids, token_table, position_table):
    raise NotImplementedError("write your pallas kernel here")


# speedup vs baseline: 1.2931x; 1.2931x over previous
import functools

import jax
import jax.numpy as jnp
from jax import lax
from jax.experimental import pallas as pl
from jax.experimental.pallas import tpu as pltpu
from jax.experimental.pallas import tpu_sc as plsc

BATCH = 4096
SEQ = 200
D = 32
TOKENS = BATCH * SEQ
NC, NS = 2, 16
NW = NC * NS
PER_W = TOKENS // NW
C = 800
NCHUNK = PER_W // C
SW = 80
NSTREAM = C // SW
REP = C // SEQ


def _embed_body(ids_hbm, tok_hbm, pos_hbm, out_hbm, idx_v, rows_v, pos_v, sem):
    wid = lax.axis_index("s") * NC + lax.axis_index("c")

    for t in range(REP):
        pltpu.sync_copy(pos_hbm, pos_v.at[pl.ds(t * SEQ, SEQ)])

    def chunk_body(g, carry):
        base = (wid * NCHUNK + g) * C
        pltpu.sync_copy(ids_hbm.at[pl.ds(base, C)], idx_v)
        descs = [
            pltpu.async_copy(
                tok_hbm.at[idx_v.at[pl.ds(j * SW, SW)]],
                rows_v.at[pl.ds(j * SW, SW)],
                sem,
            )
            for j in range(NSTREAM)
        ]
        for d in descs:
            d.wait()

        def add_body(i, c2):
            rows_v[i, pl.ds(0, 16)] = rows_v[i, pl.ds(0, 16)] + pos_v[i, pl.ds(0, 16)]
            rows_v[i, pl.ds(16, 16)] = rows_v[i, pl.ds(16, 16)] + pos_v[i, pl.ds(16, 16)]
            return c2

        lax.fori_loop(0, C, add_body, 0)
        pltpu.sync_copy(rows_v, out_hbm.at[pl.ds(base, C)])
        return carry

    lax.fori_loop(0, NCHUNK, chunk_body, 0)


@functools.partial(jax.jit, static_argnames=())
def _embed(ids_flat, token_table, position_table):
    f = pl.kernel(
        _embed_body,
        out_type=jax.ShapeDtypeStruct((TOKENS, D), jnp.float32),
        mesh=plsc.VectorSubcoreMesh(core_axis_name="c", subcore_axis_name="s"),
        scratch_types=[
            pltpu.VMEM((C,), jnp.int32),
            pltpu.VMEM((C, D), jnp.float32),
            pltpu.VMEM((C, D), jnp.float32),
            pltpu.SemaphoreType.DMA,
        ],
        compiler_params=pltpu.CompilerParams(use_tc_tiling_on_sc=False),
    )
    return f(ids_flat, token_table, position_table)


def kernel(input_ids, token_table, position_table):
    ids_flat = input_ids.reshape(-1).astype(jnp.int32)
    out = _embed(ids_flat, token_table, position_table)
    return out.reshape(BATCH, SEQ, D)
